# baseline (speedup 1.0000x reference)
import functools

import jax
import jax.numpy as jnp
from jax import lax
from jax.experimental import pallas as pl
from jax.experimental.pallas import tpu as pltpu
from jax.experimental.pallas import tpu_sc as plsc

H = 32
WPAD = 4096
NC, NS = 2, 16
NW = NC * NS
LANES = 16

G = 8
CH = 4096


def _table_body(rpb_ref, w1_ref, b1_ref, w2t_ref, out_ref):
    r = rpb_ref[...]
    w1 = w1_ref[...]
    b1 = b1_ref[...]
    w2t = w2t_ref[...]
    h = jnp.maximum(r[:, 0:1] * w1[0:1, :] + r[:, 1:2] * w1[1:2, :] + b1, 0.0)
    t = lax.dot_general(w2t, h, (((1,), (1,)), ((), ())),
                        precision=lax.Precision.HIGHEST)
    out_ref[...] = 16.0 * jax.nn.sigmoid(t)


def _build_table(rpb2, w1, b1, w2t):
    return pl.pallas_call(
        _table_body,
        out_shape=jax.ShapeDtypeStruct((H, WPAD), jnp.float32),
    )(rpb2, w1, b1, w2t)


def _sc_gather_body(table_hbm, idx_hbm, out_hbm,
                    idxv0, idxv1, ttab, obuf0, obuf1,
                    isem0, isem1, osem0, osem1):
    n_flat = out_hbm.shape[1]
    per_w = n_flat // NW
    n_chunks = per_w // CH
    wid = lax.axis_index("s") * NC + lax.axis_index("c")
    base = wid * per_w
    idxv = [idxv0, idxv1]
    isem = [isem0, isem1]
    obuf = [obuf0, obuf1]
    osem = [osem0, osem1]
    pending_out = [[], []]

    for g in range(H // G):
        pltpu.sync_copy(table_hbm.at[pl.ds(g * G * WPAD, G * WPAD)], ttab)
        in0 = pltpu.async_copy(idx_hbm.at[pl.ds(base, CH)], idxv[0], isem[0])
        pending_in = {0: in0}
        for c in range(n_chunks):
            ib, ob = c % 2, c % 2
            if c + 1 < n_chunks:
                nxt = pltpu.async_copy(
                    idx_hbm.at[pl.ds(base + (c + 1) * CH, CH)],
                    idxv[(c + 1) % 2], isem[(c + 1) % 2])
                pending_in[c + 1] = nxt
            pending_in.pop(c).wait()
            for cp in pending_out[ob]:
                cp.wait()
            pending_out[ob] = []

            @plsc.parallel_loop(0, CH // LANES, unroll=4)
            def body(v, _ib=ib, _ob=ob):
                ids = idxv[_ib][pl.ds(v * LANES, LANES)]
                for hl in range(G):
                    flat = ids + jnp.int32(hl * WPAD)
                    obuf[_ob][pl.ds(hl * CH + v * LANES, LANES)] = (
                        plsc.load_gather(ttab, [flat]))
            off = base + c * CH
            for hl in range(G):
                cp = pltpu.async_copy(obuf[ob].at[pl.ds(hl * CH, CH)],
                                      out_hbm.at[g * G + hl, pl.ds(off, CH)],
                                      osem[ob])
                pending_out[ob].append(cp)
    for lst in pending_out:
        for cp in lst:
            cp.wait()


def _make_gather(n_flat):
    mesh = plsc.VectorSubcoreMesh(core_axis_name="c", subcore_axis_name="s",
                                  num_cores=NC, num_subcores=NS)
    return pl.kernel(
        _sc_gather_body,
        out_type=jax.ShapeDtypeStruct((H, n_flat), jnp.float32),
        mesh=mesh,
        compiler_params=pltpu.CompilerParams(needs_layout_passes=False,
                                             use_tc_tiling_on_sc=True),
        scratch_types=[
            pltpu.VMEM((CH,), jnp.int32),
            pltpu.VMEM((CH,), jnp.int32),
            pltpu.VMEM((G * WPAD,), jnp.float32),
            pltpu.VMEM((G * CH,), jnp.float32),
            pltpu.VMEM((G * CH,), jnp.float32),
            pltpu.SemaphoreType.DMA,
            pltpu.SemaphoreType.DMA,
            pltpu.SemaphoreType.DMA,
            pltpu.SemaphoreType.DMA,
        ],
    )


@jax.jit
def kernel(x, rpb, W1, b1, W2, rpb_idx):
    sl = x.shape[2]
    rpb2 = rpb.reshape(-1, 2)
    rpb2 = jnp.pad(rpb2, ((0, WPAD - rpb2.shape[0]), (0, 0)))
    table = _build_table(rpb2, W1, b1.reshape(1, -1), W2.T)
    idx = rpb_idx.reshape(-1)
    out = _make_gather(idx.shape[0])(table.reshape(-1), idx)
    return out.reshape(1, H, sl, sl)


# speedup vs baseline: 1.6048x; 1.6048x over previous
import functools

import jax
import jax.numpy as jnp
from jax import lax
from jax.experimental import pallas as pl
from jax.experimental.pallas import tpu as pltpu
from jax.experimental.pallas import tpu_sc as plsc

H = 32
WPAD = 4096
NC, NS = 2, 16
NW = NC * NS
LANES = 16

G = 4
CH = 8192


def _table_body(rpb_ref, w1_ref, b1_ref, w2t_ref, out_ref):
    r = rpb_ref[...]
    w1 = w1_ref[...]
    b1 = b1_ref[...]
    w2t = w2t_ref[...]
    h = jnp.maximum(r[:, 0:1] * w1[0:1, :] + r[:, 1:2] * w1[1:2, :] + b1, 0.0)
    t = lax.dot_general(w2t, h, (((1,), (1,)), ((), ())),
                        precision=lax.Precision.HIGHEST)
    out_ref[...] = 16.0 * jax.nn.sigmoid(t)


def _build_table(rpb2, w1, b1, w2t):
    return pl.pallas_call(
        _table_body,
        out_shape=jax.ShapeDtypeStruct((H, WPAD), jnp.float32),
    )(rpb2, w1, b1, w2t)


def _sc_gather_body(table_hbm, idx_hbm, out_hbm,
                    idxv0, idxv1, ttab, obuf0, obuf1,
                    isem0, isem1, osem0, osem1):
    sl = out_hbm.shape[2]
    n_flat = sl * sl
    rows_per_chunk = CH // sl
    per_w = n_flat // NW
    n_chunks = per_w // CH
    wid = lax.axis_index("s") * NC + lax.axis_index("c")
    base = wid * per_w
    idxv = [idxv0, idxv1]
    isem = [isem0, isem1]
    obuf = [obuf0, obuf1]
    osem = [osem0, osem1]
    pending_out = [[], []]

    for g in range(H // G):
        pltpu.sync_copy(table_hbm.at[pl.ds(g * G * WPAD, G * WPAD)], ttab)
        in0 = pltpu.async_copy(idx_hbm.at[pl.ds(base, CH)], idxv[0], isem[0])
        pending_in = {0: in0}
        for c in range(n_chunks):
            ib, ob = c % 2, c % 2
            if c + 1 < n_chunks:
                nxt = pltpu.async_copy(
                    idx_hbm.at[pl.ds(base + (c + 1) * CH, CH)],
                    idxv[(c + 1) % 2], isem[(c + 1) % 2])
                pending_in[c + 1] = nxt
            pending_in.pop(c).wait()
            for cp in pending_out[ob]:
                cp.wait()
            pending_out[ob] = []

            vec_per_row = sl // LANES
            vshift = vec_per_row.bit_length() - 1

            @plsc.parallel_loop(0, CH // LANES, unroll=4)
            def body(v, _ib=ib, _ob=ob):
                ids = idxv[_ib][pl.ds(v * LANES, LANES)]
                srow = jax.lax.shift_right_logical(v, vshift)
                scol = (v & (vec_per_row - 1)) * LANES
                for hl in range(G):
                    flat = ids + jnp.int32(hl * WPAD)
                    obuf[_ob][hl * rows_per_chunk + srow,
                              pl.ds(scol, LANES)] = (
                        plsc.load_gather(ttab, [flat]))
            off = base + c * CH
            row0 = pl.multiple_of(off // sl, rows_per_chunk)
            for hl in range(G):
                cp = pltpu.async_copy(
                    obuf[ob].at[pl.ds(hl * rows_per_chunk, rows_per_chunk)],
                    out_hbm.at[0, g * G + hl, pl.ds(row0, rows_per_chunk), :],
                    osem[ob])
                pending_out[ob].append(cp)
    for lst in pending_out:
        for cp in lst:
            cp.wait()


def _make_gather(sl):
    mesh = plsc.VectorSubcoreMesh(core_axis_name="c", subcore_axis_name="s",
                                  num_cores=NC, num_subcores=NS)
    return pl.kernel(
        _sc_gather_body,
        out_type=jax.ShapeDtypeStruct((1, H, sl, sl), jnp.float32),
        mesh=mesh,
        compiler_params=pltpu.CompilerParams(needs_layout_passes=False,
                                             use_tc_tiling_on_sc=True),
        scratch_types=[
            pltpu.VMEM((CH,), jnp.int32),
            pltpu.VMEM((CH,), jnp.int32),
            pltpu.VMEM((G * WPAD,), jnp.float32),
            pltpu.VMEM((G * (CH // sl), sl), jnp.float32),
            pltpu.VMEM((G * (CH // sl), sl), jnp.float32),
            pltpu.SemaphoreType.DMA,
            pltpu.SemaphoreType.DMA,
            pltpu.SemaphoreType.DMA,
            pltpu.SemaphoreType.DMA,
        ],
    )


@jax.jit
def kernel(x, rpb, W1, b1, W2, rpb_idx):
    sl = x.shape[2]
    rpb2 = rpb.reshape(-1, 2)
    rpb2 = jnp.pad(rpb2, ((0, WPAD - rpb2.shape[0]), (0, 0)))
    table = _build_table(rpb2, W1, b1.reshape(1, -1), W2.T)
    idx = rpb_idx.reshape(-1)
    return _make_gather(sl)(table.reshape(-1), idx)


# speedup vs baseline: 1.8758x; 1.1689x over previous
import functools

import jax
import jax.numpy as jnp
from jax import lax
from jax.experimental import pallas as pl
from jax.experimental.pallas import tpu as pltpu
from jax.experimental.pallas import tpu_sc as plsc

H = 32
WPAD = 4096
NC, NS = 2, 16
NW = NC * NS
LANES = 16

G = 8
BR = 8
BC = 512


def _table_body(rpb_ref, w1_ref, b1_ref, w2t_ref, out_ref):
    r = rpb_ref[...]
    w1 = w1_ref[...]
    b1 = b1_ref[...]
    w2t = w2t_ref[...]
    h = jnp.maximum(r[:, 0:1] * w1[0:1, :] + r[:, 1:2] * w1[1:2, :] + b1, 0.0)
    t = lax.dot_general(w2t, h, (((1,), (1,)), ((), ())),
                        precision=lax.Precision.HIGHEST)
    out_ref[...] = 16.0 * jax.nn.sigmoid(t)


def _build_table(rpb2, w1, b1, w2t):
    return pl.pallas_call(
        _table_body,
        out_shape=jax.ShapeDtypeStruct((H, WPAD), jnp.float32),
    )(rpb2, w1, b1, w2t)


def _sc_gather_body(table_hbm, idx_hbm, out_hbm,
                    idxv0, idxv1, ttab, obuf0, obuf1,
                    isem0, isem1, osem0, osem1):
    sl = out_hbm.shape[2]
    rows_per_w = sl // NW
    n_chunks = (rows_per_w // BR) * (sl // BC)
    cols_per_row = sl // BC
    wid = lax.axis_index("s") * NC + lax.axis_index("c")
    idxv = [idxv0, idxv1]
    isem = [isem0, isem1]
    obuf = [obuf0, obuf1]
    osem = [osem0, osem1]
    pending_out = [[], []]

    def block_origin(c):
        r0 = pl.multiple_of(wid * rows_per_w + (c // cols_per_row) * BR, BR)
        c0 = (c % cols_per_row) * BC
        return r0, c0

    for g in range(H // G):
        pltpu.sync_copy(table_hbm.at[pl.ds(g * G * WPAD, G * WPAD)], ttab)
        r0, c0 = block_origin(0)
        in0 = pltpu.async_copy(
            idx_hbm.at[pl.ds(r0, BR), pl.ds(c0, BC)], idxv[0], isem[0])
        pending_in = {0: in0}
        for c in range(n_chunks):
            ib, ob = c % 2, c % 2
            if c + 1 < n_chunks:
                rn, cn = block_origin(c + 1)
                nxt = pltpu.async_copy(
                    idx_hbm.at[pl.ds(rn, BR), pl.ds(cn, BC)],
                    idxv[(c + 1) % 2], isem[(c + 1) % 2])
                pending_in[c + 1] = nxt
            pending_in.pop(c).wait()
            for cp in pending_out[ob]:
                cp.wait()
            pending_out[ob] = []

            vec_per_row = BC // LANES
            vshift = vec_per_row.bit_length() - 1

            @plsc.parallel_loop(0, (BR * BC) // LANES, unroll=4)
            def body(v, _ib=ib, _ob=ob):
                srow = jax.lax.shift_right_logical(v, vshift)
                scol = (v & (vec_per_row - 1)) * LANES
                ids = idxv[_ib][srow, pl.ds(scol, LANES)]
                for hl in range(G):
                    flat = ids + jnp.int32(hl * WPAD)
                    obuf[_ob][hl * BR + srow, pl.ds(scol, LANES)] = (
                        plsc.load_gather(ttab, [flat]))

            r0, c0 = block_origin(c)
            for hl in range(G):
                cp = pltpu.async_copy(
                    obuf[ob].at[pl.ds(hl * BR, BR)],
                    out_hbm.at[0, g * G + hl, pl.ds(r0, BR), pl.ds(c0, BC)],
                    osem[ob])
                pending_out[ob].append(cp)
    for lst in pending_out:
        for cp in lst:
            cp.wait()


def _make_gather(sl):
    mesh = plsc.VectorSubcoreMesh(core_axis_name="c", subcore_axis_name="s",
                                  num_cores=NC, num_subcores=NS)
    return pl.kernel(
        _sc_gather_body,
        out_type=jax.ShapeDtypeStruct((1, H, sl, sl), jnp.float32),
        mesh=mesh,
        compiler_params=pltpu.CompilerParams(needs_layout_passes=False,
                                             use_tc_tiling_on_sc=True),
        scratch_types=[
            pltpu.VMEM((BR, BC), jnp.int32),
            pltpu.VMEM((BR, BC), jnp.int32),
            pltpu.VMEM((G * WPAD,), jnp.float32),
            pltpu.VMEM((G * BR, BC), jnp.float32),
            pltpu.VMEM((G * BR, BC), jnp.float32),
            pltpu.SemaphoreType.DMA,
            pltpu.SemaphoreType.DMA,
            pltpu.SemaphoreType.DMA,
            pltpu.SemaphoreType.DMA,
        ],
    )


@jax.jit
def kernel(x, rpb, W1, b1, W2, rpb_idx):
    sl = x.shape[2]
    rpb2 = rpb.reshape(-1, 2)
    rpb2 = jnp.pad(rpb2, ((0, WPAD - rpb2.shape[0]), (0, 0)))
    table = _build_table(rpb2, W1, b1.reshape(1, -1), W2.T)
    return _make_gather(sl)(table.reshape(-1), rpb_idx)
